# Initial kernel scaffold; baseline (speedup 1.0000x reference)
import jax
import jax.numpy as jnp
from jax.experimental import pallas as pl


def kernel(emb_table, edge_index, n_id):
    raise NotImplementedError("write your pallas kernel here")


# speedup vs baseline: 6.8379x; 6.8379x over previous
import functools

import jax
import jax.numpy as jnp
from jax import lax
from jax.experimental import pallas as pl
from jax.experimental.pallas import tpu as pltpu
from jax.experimental.pallas import tpu_sc as plsc

N = 50000
E = 800000
D = 64
HALF = 32
NC = 2
NS = 16
NPAD = 51200
EPAD = 802816
EB = 128
ROWS_PER_TILE = NPAD // NS
ROW_CHUNKS = ROWS_PER_TILE // 128
E_PER_TILE_ALL = EPAD // NS
E_BATCHES_ALL = E_PER_TILE_ALL // EB
E_PER_TILE_HALF = EPAD // (NC * NS)
E_BATCHES_HALF = E_PER_TILE_HALF // EB
GATHER_ROWS = NPAD // (NC * NS)
GB = 64
GATHER_BATCHES = GATHER_ROWS // GB

_mesh = lambda: plsc.VectorSubcoreMesh(core_axis_name="c", subcore_axis_name="s")


def _deg_body(col_hbm, ones_hbm, zeros_hbm, deg_hbm, colv, onesv, sem, acc):
    c = lax.axis_index("c")
    s = lax.axis_index("s")
    zbase = s * ROWS_PER_TILE

    @pl.loop(0, ROW_CHUNKS)
    def _(k):
        pltpu.sync_copy(zeros_hbm, acc.at[pl.ds(zbase + 128 * k, 128)])

    pltpu.sync_copy(ones_hbm, onesv)
    plsc.subcore_barrier()

    e0 = c * (EPAD // 2) + s * E_PER_TILE_HALF

    @pl.loop(0, E_BATCHES_HALF)
    def _(b):
        pltpu.sync_copy(col_hbm.at[pl.ds(e0 + EB * b, EB)], colv)
        pltpu.sync_copy(onesv, acc.at[colv], add=True)

    plsc.subcore_barrier()
    obase = c * NPAD + s * ROWS_PER_TILE

    @pl.loop(0, ROW_CHUNKS)
    def _(k):
        pltpu.sync_copy(acc.at[pl.ds(zbase + 128 * k, 128)],
                        deg_hbm.at[pl.ds(obase + 128 * k, 128)])


_deg_call = pl.kernel(
    _deg_body,
    out_type=jax.ShapeDtypeStruct((NC * NPAD, 8), jnp.float32),
    mesh=_mesh(),
    compiler_params=pltpu.CompilerParams(use_tc_tiling_on_sc=False),
    scratch_types=[
        pltpu.VMEM((EB,), jnp.int32),
        pltpu.VMEM((EB, 8), jnp.float32),
        pltpu.SemaphoreType.DMA,
        pltpu.VMEM_SHARED((NPAD, 8), jnp.float32),
    ],
)


def _gather_body(emb_hbm, nid_hbm, x0_hbm, idxv, rowsv, sem):
    c = lax.axis_index("c")
    s = lax.axis_index("s")
    wid = s * NC + c

    @pl.loop(0, GATHER_BATCHES)
    def _(k):
        off = wid * GATHER_ROWS + GB * k
        pltpu.sync_copy(nid_hbm.at[pl.ds(off, GB)], idxv)
        pltpu.async_copy(emb_hbm.at[idxv], rowsv, sem).wait()
        pltpu.sync_copy(rowsv, x0_hbm.at[pl.ds(off, GB)])


_gather_call = pl.kernel(
    _gather_body,
    out_type=jax.ShapeDtypeStruct((NPAD, D), jnp.float32),
    mesh=_mesh(),
    compiler_params=pltpu.CompilerParams(use_tc_tiling_on_sc=False),
    scratch_types=[
        pltpu.VMEM((GB,), jnp.int32),
        pltpu.VMEM((GB, D), jnp.float32),
        pltpu.SemaphoreType.DMA,
    ],
)


def _segsum_body(h_hbm, row_hbm, col_hbm, zeros_hbm, s_hbm,
                 rowv, colv, gbuf, sem, acc):
    c = lax.axis_index("c")
    s = lax.axis_index("s")
    zbase = s * ROWS_PER_TILE

    @pl.loop(0, ROW_CHUNKS)
    def _(k):
        pltpu.sync_copy(zeros_hbm, acc.at[pl.ds(zbase + 128 * k, 128)])

    plsc.subcore_barrier()

    hoff = c * NPAD
    e0 = s * E_PER_TILE_ALL

    @pl.loop(0, E_BATCHES_ALL)
    def _(b):
        off = e0 + EB * b
        pltpu.sync_copy(row_hbm.at[pl.ds(off, EB)], rowv)
        pltpu.sync_copy(col_hbm.at[pl.ds(off, EB)], colv)
        for q in range(EB // 16):
            rowv[pl.ds(q * 16, 16)] = rowv[pl.ds(q * 16, 16)] + hoff
        pltpu.async_copy(h_hbm.at[rowv], gbuf, sem).wait()
        pltpu.sync_copy(gbuf, acc.at[colv], add=True)

    plsc.subcore_barrier()
    obase = c * NPAD + s * ROWS_PER_TILE

    @pl.loop(0, ROW_CHUNKS)
    def _(k):
        pltpu.sync_copy(acc.at[pl.ds(zbase + 128 * k, 128)],
                        s_hbm.at[pl.ds(obase + 128 * k, 128)])


_segsum_call = pl.kernel(
    _segsum_body,
    out_type=jax.ShapeDtypeStruct((NC * NPAD, HALF), jnp.float32),
    mesh=_mesh(),
    compiler_params=pltpu.CompilerParams(use_tc_tiling_on_sc=False),
    scratch_types=[
        pltpu.VMEM((EB,), jnp.int32),
        pltpu.VMEM((EB,), jnp.int32),
        pltpu.VMEM((EB, HALF), jnp.float32),
        pltpu.SemaphoreType.DMA,
        pltpu.VMEM_SHARED((NPAD, HALF), jnp.float32),
    ],
)


def _scale0_body(x0_ref, dega_ref, degb_ref, ha_ref, hb_ref):
    deg = dega_ref[...] + degb_ref[...]
    dis = jnp.where(deg > 0, lax.rsqrt(deg), 0.0)[:, :1]
    ha_ref[...] = x0_ref[:, :HALF] * dis
    hb_ref[...] = x0_ref[:, HALF:] * dis


def _scale1_body(s1a_ref, s1b_ref, dega_ref, degb_ref, ha_ref, hb_ref):
    deg = dega_ref[...] + degb_ref[...]
    inv = jnp.where(deg > 0, 1.0 / deg, 0.0)[:, :1]
    ha_ref[...] = s1a_ref[...] * inv
    hb_ref[...] = s1b_ref[...] * inv


def _final_body(x0_ref, s1a_ref, s1b_ref, s2a_ref, s2b_ref,
                dega_ref, degb_ref, z_ref):
    deg = dega_ref[...] + degb_ref[...]
    dis = jnp.where(deg > 0, lax.rsqrt(deg), 0.0)[:, :1]
    sa = (s1a_ref[...] + s2a_ref[...]) * dis
    sb = (s1b_ref[...] + s2b_ref[...]) * dis
    z_ref[...] = (x0_ref[...] + jnp.concatenate([sa, sb], axis=1)) * (1.0 / 3.0)


_RS = 256
_NB = NPAD // _RS
_half_out = (jax.ShapeDtypeStruct((NPAD, HALF), jnp.float32),) * 2
_scale0_call = pl.pallas_call(
    _scale0_body,
    grid=(_NB,),
    in_specs=[
        pl.BlockSpec((_RS, D), lambda i: (i, 0)),
        pl.BlockSpec((_RS, 8), lambda i: (i, 0)),
        pl.BlockSpec((_RS, 8), lambda i: (_NB + i, 0)),
    ],
    out_specs=[pl.BlockSpec((_RS, HALF), lambda i: (i, 0))] * 2,
    out_shape=_half_out,
)
_scale1_call = pl.pallas_call(
    _scale1_body,
    grid=(_NB,),
    in_specs=[
        pl.BlockSpec((_RS, HALF), lambda i: (i, 0)),
        pl.BlockSpec((_RS, HALF), lambda i: (_NB + i, 0)),
        pl.BlockSpec((_RS, 8), lambda i: (i, 0)),
        pl.BlockSpec((_RS, 8), lambda i: (_NB + i, 0)),
    ],
    out_specs=[pl.BlockSpec((_RS, HALF), lambda i: (i, 0))] * 2,
    out_shape=_half_out,
)
_RF = 80
_NF = NPAD // _RF
_final_call = pl.pallas_call(
    _final_body,
    grid=(N // _RF,),
    in_specs=[
        pl.BlockSpec((_RF, D), lambda i: (i, 0)),
        pl.BlockSpec((_RF, HALF), lambda i: (i, 0)),
        pl.BlockSpec((_RF, HALF), lambda i: (_NF + i, 0)),
        pl.BlockSpec((_RF, HALF), lambda i: (i, 0)),
        pl.BlockSpec((_RF, HALF), lambda i: (_NF + i, 0)),
        pl.BlockSpec((_RF, 8), lambda i: (i, 0)),
        pl.BlockSpec((_RF, 8), lambda i: (_NF + i, 0)),
    ],
    out_specs=pl.BlockSpec((_RF, D), lambda i: (i, 0)),
    out_shape=jax.ShapeDtypeStruct((N, D), jnp.float32),
)


def kernel(emb_table, edge_index, n_id):
    row = edge_index[0].astype(jnp.int32)
    col = edge_index[1].astype(jnp.int32)
    rowp = jnp.concatenate([row, jnp.zeros((EPAD - E,), jnp.int32)])
    colp = jnp.concatenate([col, jnp.full((EPAD - E,), N, jnp.int32)])
    nidp = jnp.concatenate([n_id.astype(jnp.int32),
                            jnp.zeros((NPAD - N,), jnp.int32)])
    zeros8 = jnp.zeros((EB, 8), jnp.float32)
    ones8 = jnp.ones((EB, 8), jnp.float32)
    zeros32 = jnp.zeros((128, HALF), jnp.float32)

    deg = _deg_call(colp, ones8, zeros8)
    x0 = _gather_call(emb_table, nidp)
    h0a, h0b = _scale0_call(x0, deg, deg)
    s1 = _segsum_call(jnp.concatenate([h0a, h0b]), rowp, colp, zeros32)
    h1a, h1b = _scale1_call(s1, s1, deg, deg)
    s2 = _segsum_call(jnp.concatenate([h1a, h1b]), rowp, colp, zeros32)
    return _final_call(x0, s1, s1, s2, s2, deg, deg)


# speedup vs baseline: 11.3647x; 1.6620x over previous
import jax
import jax.numpy as jnp
from jax import lax
from jax.experimental import pallas as pl
from jax.experimental.pallas import tpu as pltpu
from jax.experimental.pallas import tpu_sc as plsc

N = 50000
E = 800000
D = 64
HALF = 32
NC = 2
NS = 16
NPAD = 51200
NACC = 50176
EPAD = 811008
EB = 128
ECHUNKS = EPAD // EB
ROWS_PER_TILE = NACC // NS
K = 3
NG = ECHUNKS // (NS * K)
CPT = NG * K
DEG_CPT = ECHUNKS // (NC * NS)
KD = 6
GATHER_ROWS = NACC // (NC * NS)

_mesh = lambda: plsc.VectorSubcoreMesh(core_axis_name="c", subcore_axis_name="s")
_params = lambda: pltpu.CompilerParams(use_tc_tiling_on_sc=False)


def _deg_body(cols2d, ones_hbm, zeros_hbm, deg_hbm, colv2, onesv, dsem, acc):
    c = lax.axis_index("c")
    s = lax.axis_index("s")
    zbase = s * ROWS_PER_TILE
    pltpu.sync_copy(zeros_hbm, acc.at[pl.ds(zbase, ROWS_PER_TILE)])
    pltpu.sync_copy(ones_hbm, onesv)
    pltpu.sync_copy(cols2d.at[pl.ds(c * (ECHUNKS // NC) + s * DEG_CPT,
                                    DEG_CPT)], colv2)
    plsc.subcore_barrier()

    @pl.loop(0, DEG_CPT // KD)
    def _(i):
        descs = [pltpu.async_copy(onesv, acc.at[colv2.at[i * KD + j]], dsem,
                                  add=True) for j in range(KD)]
        for d in descs:
            d.wait()

    plsc.subcore_barrier()
    pltpu.sync_copy(acc.at[pl.ds(zbase, ROWS_PER_TILE)],
                    deg_hbm.at[pl.ds(c * NPAD + zbase, ROWS_PER_TILE)])


_deg_call = pl.kernel(
    _deg_body,
    out_type=jax.ShapeDtypeStruct((NC * NPAD, 8), jnp.float32),
    mesh=_mesh(),
    compiler_params=_params(),
    scratch_types=[
        pltpu.VMEM((DEG_CPT, EB), jnp.int32),
        pltpu.VMEM((EB, 8), jnp.float32),
        pltpu.SemaphoreType.DMA,
        pltpu.VMEM_SHARED((NACC, 8), jnp.float32),
    ],
)


def _gather_body(emb_hbm, nid_hbm, x0_hbm, idxv, gbuf, sem):
    c = lax.axis_index("c")
    s = lax.axis_index("s")
    off = (s * NC + c) * GATHER_ROWS
    pltpu.sync_copy(nid_hbm.at[pl.ds(off, GATHER_ROWS)], idxv)
    descs = []
    for j in range(GATHER_ROWS // EB):
        descs.append(pltpu.async_copy(
            emb_hbm.at[idxv.at[pl.ds(j * EB, EB)]],
            gbuf.at[pl.ds(j * EB, EB)], sem))
    rem = GATHER_ROWS % EB
    if rem:
        base = (GATHER_ROWS // EB) * EB
        descs.append(pltpu.async_copy(
            emb_hbm.at[idxv.at[pl.ds(base, rem)]],
            gbuf.at[pl.ds(base, rem)], sem))
    for d in descs:
        d.wait()
    pltpu.sync_copy(gbuf, x0_hbm.at[pl.ds(off, GATHER_ROWS)])


_gather_call = pl.kernel(
    _gather_body,
    out_type=jax.ShapeDtypeStruct((NPAD, D), jnp.float32),
    mesh=_mesh(),
    compiler_params=_params(),
    scratch_types=[
        pltpu.VMEM((GATHER_ROWS,), jnp.int32),
        pltpu.VMEM((GATHER_ROWS, D), jnp.float32),
        pltpu.SemaphoreType.DMA,
    ],
)


def _segsum_body(ha_hbm, hb_hbm, rows2d, cols2d, zeros_hbm, s_hbm,
                 rowv, colv, gbuf,
                 isem0, isem1, isem2, isem3, gsem, ssem0, ssem1, ssem2, ssem3,
                 acc):
    c = lax.axis_index("c")
    s = lax.axis_index("s")
    isems = (isem0, isem1, isem2, isem3)
    ssems = (ssem0, ssem1, ssem2, ssem3)
    zbase = s * ROWS_PER_TILE
    cbase = s * CPT

    pltpu.sync_copy(zeros_hbm, acc.at[pl.ds(zbase, ROWS_PER_TILE)])
    plsc.subcore_barrier()

    def issue_idx(g, rset):
        pltpu.async_copy(rows2d.at[pl.ds(cbase + g * K, K)],
                         rowv.at[rset], isems[rset])
        pltpu.async_copy(cols2d.at[pl.ds(cbase + g * K, K)],
                         colv.at[rset], isems[rset])

    def wait_idx(rset):
        pltpu.make_async_copy(rows2d.at[pl.ds(0, K)], rowv.at[rset],
                              isems[rset]).wait()
        pltpu.make_async_copy(cols2d.at[pl.ds(0, K)], colv.at[rset],
                              isems[rset]).wait()

    def drain_scatters(rset):
        for j in range(K):
            pltpu.make_async_copy(ha_hbm.at[pl.ds(0, EB)], gbuf.at[0, j],
                                  ssems[rset]).wait()

    def body(g, rset, p, drain, prefetch):
        if drain:
            drain_scatters((rset + 2) % 4)
        wait_idx(rset)

        @pl.when(c == 0)
        def _():
            for j in range(K):
                pltpu.async_copy(ha_hbm.at[rowv.at[rset, j]],
                                 gbuf.at[p, j], gsem)

        @pl.when(c == 1)
        def _():
            for j in range(K):
                pltpu.async_copy(hb_hbm.at[rowv.at[rset, j]],
                                 gbuf.at[p, j], gsem)

        for j in range(K):
            pltpu.make_async_copy(ha_hbm.at[pl.ds(0, EB)], gbuf.at[0, j],
                                  gsem).wait()
        for j in range(K):
            pltpu.async_copy(gbuf.at[p, j], acc.at[colv.at[rset, j]],
                             ssems[rset], add=True)
        if prefetch:
            issue_idx(g + 1, (rset + 1) % 4)

    issue_idx(0, 0)
    for g0 in range(4):
        body(g0, g0 % 4, g0 % 2, drain=(g0 >= 2), prefetch=True)

    @pl.loop(1, NG // 4 - 1)
    def _(sg):
        for r in range(4):
            body(sg * 4 + r, r, r % 2, drain=True, prefetch=True)

    for g1 in range(NG - 4, NG):
        body(g1, g1 % 4, g1 % 2, drain=True, prefetch=(g1 < NG - 1))
    drain_scatters((NG - 2) % 4)
    drain_scatters((NG - 1) % 4)

    plsc.subcore_barrier()
    pltpu.sync_copy(acc.at[pl.ds(zbase, ROWS_PER_TILE)],
                    s_hbm.at[pl.ds(c * NPAD + zbase, ROWS_PER_TILE)])


_segsum_call = pl.kernel(
    _segsum_body,
    out_type=jax.ShapeDtypeStruct((NC * NPAD, HALF), jnp.float32),
    mesh=_mesh(),
    compiler_params=_params(),
    scratch_types=[
        pltpu.VMEM((4, K, EB), jnp.int32),
        pltpu.VMEM((4, K, EB), jnp.int32),
        pltpu.VMEM((2, K, EB, HALF), jnp.float32),
        pltpu.SemaphoreType.DMA,
        pltpu.SemaphoreType.DMA,
        pltpu.SemaphoreType.DMA,
        pltpu.SemaphoreType.DMA,
        pltpu.SemaphoreType.DMA,
        pltpu.SemaphoreType.DMA,
        pltpu.SemaphoreType.DMA,
        pltpu.SemaphoreType.DMA,
        pltpu.SemaphoreType.DMA,
        pltpu.VMEM_SHARED((NACC, HALF), jnp.float32),
    ],
)


def _scale0_body(x0_ref, dega_ref, degb_ref, ha_ref, hb_ref):
    deg = dega_ref[...] + degb_ref[...]
    dis = jnp.where(deg > 0, lax.rsqrt(deg), 0.0)[:, :1]
    ha_ref[...] = x0_ref[:, :HALF] * dis
    hb_ref[...] = x0_ref[:, HALF:] * dis


def _scale1_body(s1a_ref, s1b_ref, dega_ref, degb_ref, ha_ref, hb_ref):
    deg = dega_ref[...] + degb_ref[...]
    inv = jnp.where(deg > 0, 1.0 / deg, 0.0)[:, :1]
    ha_ref[...] = s1a_ref[...] * inv
    hb_ref[...] = s1b_ref[...] * inv


def _final_body(x0_ref, s1a_ref, s1b_ref, s2a_ref, s2b_ref,
                dega_ref, degb_ref, z_ref):
    deg = dega_ref[...] + degb_ref[...]
    dis = jnp.where(deg > 0, lax.rsqrt(deg), 0.0)[:, :1]
    sa = (s1a_ref[...] + s2a_ref[...]) * dis
    sb = (s1b_ref[...] + s2b_ref[...]) * dis
    z_ref[...] = (x0_ref[...] + jnp.concatenate([sa, sb], axis=1)) * (1.0 / 3.0)


_RS = 256
_NB = NPAD // _RS
_half_out = (jax.ShapeDtypeStruct((NPAD, HALF), jnp.float32),) * 2
_scale0_call = pl.pallas_call(
    _scale0_body,
    grid=(_NB,),
    in_specs=[
        pl.BlockSpec((_RS, D), lambda i: (i, 0)),
        pl.BlockSpec((_RS, 8), lambda i: (i, 0)),
        pl.BlockSpec((_RS, 8), lambda i: (_NB + i, 0)),
    ],
    out_specs=[pl.BlockSpec((_RS, HALF), lambda i: (i, 0))] * 2,
    out_shape=_half_out,
)
_scale1_call = pl.pallas_call(
    _scale1_body,
    grid=(_NB,),
    in_specs=[
        pl.BlockSpec((_RS, HALF), lambda i: (i, 0)),
        pl.BlockSpec((_RS, HALF), lambda i: (_NB + i, 0)),
        pl.BlockSpec((_RS, 8), lambda i: (i, 0)),
        pl.BlockSpec((_RS, 8), lambda i: (_NB + i, 0)),
    ],
    out_specs=[pl.BlockSpec((_RS, HALF), lambda i: (i, 0))] * 2,
    out_shape=_half_out,
)
_RF = 80
_NF = NPAD // _RF
_final_call = pl.pallas_call(
    _final_body,
    grid=(N // _RF,),
    in_specs=[
        pl.BlockSpec((_RF, D), lambda i: (i, 0)),
        pl.BlockSpec((_RF, HALF), lambda i: (i, 0)),
        pl.BlockSpec((_RF, HALF), lambda i: (_NF + i, 0)),
        pl.BlockSpec((_RF, HALF), lambda i: (i, 0)),
        pl.BlockSpec((_RF, HALF), lambda i: (_NF + i, 0)),
        pl.BlockSpec((_RF, 8), lambda i: (i, 0)),
        pl.BlockSpec((_RF, 8), lambda i: (_NF + i, 0)),
    ],
    out_specs=pl.BlockSpec((_RF, D), lambda i: (i, 0)),
    out_shape=jax.ShapeDtypeStruct((N, D), jnp.float32),
)


def kernel(emb_table, edge_index, n_id):
    row = edge_index[0].astype(jnp.int32)
    col = edge_index[1].astype(jnp.int32)
    rows2d = jnp.concatenate([row, jnp.zeros((EPAD - E,), jnp.int32)]
                             ).reshape(ECHUNKS, EB)
    cols2d = jnp.concatenate([col, jnp.full((EPAD - E,), N, jnp.int32)]
                             ).reshape(ECHUNKS, EB)
    nidp = jnp.concatenate([n_id.astype(jnp.int32),
                            jnp.zeros((NACC - N,), jnp.int32)])
    zeros8 = jnp.zeros((ROWS_PER_TILE, 8), jnp.float32)
    ones8 = jnp.ones((EB, 8), jnp.float32)
    zeros32 = jnp.zeros((ROWS_PER_TILE, HALF), jnp.float32)

    deg = _deg_call(cols2d, ones8, zeros8)
    x0 = _gather_call(emb_table, nidp)
    h0a, h0b = _scale0_call(x0, deg, deg)
    s1 = _segsum_call(h0a, h0b, rows2d, cols2d, zeros32)
    h1a, h1b = _scale1_call(s1, s1, deg, deg)
    s2 = _segsum_call(h1a, h1b, rows2d, cols2d, zeros32)
    return _final_call(x0, s1, s1, s2, s2, deg, deg)


# speedup vs baseline: 15.0057x; 1.3204x over previous
import jax
import jax.numpy as jnp
from jax import lax
from jax.experimental import pallas as pl
from jax.experimental.pallas import tpu as pltpu
from jax.experimental.pallas import tpu_sc as plsc

N = 50000
E = 800000
D = 64
HALF = 32
NC = 2
NS = 16
NPAD = 51200
NACC = 50176
EPAD = 811008
EB = 128
ECHUNKS = EPAD // EB
ROWS_PER_TILE = NACC // NS
K = 3
NG = ECHUNKS // (NS * K)
CPT = NG * K
DEG_CPT = ECHUNKS // (NC * NS)
KD = 6
GATHER_ROWS = NACC // (NC * NS)

_mesh = lambda: plsc.VectorSubcoreMesh(core_axis_name="c", subcore_axis_name="s")
_params = lambda: pltpu.CompilerParams(use_tc_tiling_on_sc=False)


def _deg_body(cols2d, ones_hbm, zeros_hbm, deg_hbm, colv2, onesv, dsem, acc):
    c = lax.axis_index("c")
    s = lax.axis_index("s")
    zbase = s * ROWS_PER_TILE
    pltpu.sync_copy(zeros_hbm, acc.at[pl.ds(zbase, ROWS_PER_TILE)])
    pltpu.sync_copy(ones_hbm, onesv)
    pltpu.sync_copy(cols2d.at[pl.ds(c * (ECHUNKS // NC) + s * DEG_CPT,
                                    DEG_CPT)], colv2)
    plsc.subcore_barrier()

    @pl.loop(0, DEG_CPT // KD)
    def _(i):
        descs = [pltpu.async_copy(onesv, acc.at[colv2.at[i * KD + j]], dsem,
                                  add=True) for j in range(KD)]
        for d in descs:
            d.wait()

    plsc.subcore_barrier()
    pltpu.sync_copy(acc.at[pl.ds(zbase, ROWS_PER_TILE)],
                    deg_hbm.at[pl.ds(c * NPAD + zbase, ROWS_PER_TILE)])


_deg_call = pl.kernel(
    _deg_body,
    out_type=jax.ShapeDtypeStruct((NC * NPAD, 8), jnp.float32),
    mesh=_mesh(),
    compiler_params=_params(),
    scratch_types=[
        pltpu.VMEM((DEG_CPT, EB), jnp.int32),
        pltpu.VMEM((EB, 8), jnp.float32),
        pltpu.SemaphoreType.DMA,
        pltpu.VMEM_SHARED((NACC, 8), jnp.float32),
    ],
)


def _gather_body(emb_hbm, nid_hbm, x0_hbm, idxv, gbuf, sem):
    c = lax.axis_index("c")
    s = lax.axis_index("s")
    off = (s * NC + c) * GATHER_ROWS
    pltpu.sync_copy(nid_hbm.at[pl.ds(off, GATHER_ROWS)], idxv)
    descs = []
    for j in range(GATHER_ROWS // EB):
        descs.append(pltpu.async_copy(
            emb_hbm.at[idxv.at[pl.ds(j * EB, EB)]],
            gbuf.at[pl.ds(j * EB, EB)], sem))
    rem = GATHER_ROWS % EB
    if rem:
        base = (GATHER_ROWS // EB) * EB
        descs.append(pltpu.async_copy(
            emb_hbm.at[idxv.at[pl.ds(base, rem)]],
            gbuf.at[pl.ds(base, rem)], sem))
    for d in descs:
        d.wait()
    pltpu.sync_copy(gbuf, x0_hbm.at[pl.ds(off, GATHER_ROWS)])


_gather_call = pl.kernel(
    _gather_body,
    out_type=jax.ShapeDtypeStruct((NPAD, D), jnp.float32),
    mesh=_mesh(),
    compiler_params=_params(),
    scratch_types=[
        pltpu.VMEM((GATHER_ROWS,), jnp.int32),
        pltpu.VMEM((GATHER_ROWS, D), jnp.float32),
        pltpu.SemaphoreType.DMA,
    ],
)


def _segsum_body(ha_hbm, hb_hbm, rows2d, cols2d, zeros_hbm, s_hbm,
                 rowv, colv, gbuf,
                 isem0, isem1, isem2, isem3, gsem, ssem0, ssem1, ssem2, ssem3,
                 acc):
    c = lax.axis_index("c")
    s = lax.axis_index("s")
    isems = (isem0, isem1, isem2, isem3)
    ssems = (ssem0, ssem1, ssem2, ssem3)
    zbase = s * ROWS_PER_TILE
    cbase = s * CPT

    pltpu.sync_copy(zeros_hbm, acc.at[pl.ds(zbase, ROWS_PER_TILE)])
    plsc.subcore_barrier()

    def issue_idx(g, rset):
        pltpu.async_copy(rows2d.at[pl.ds(cbase + g * K, K)],
                         rowv.at[rset], isems[rset])
        pltpu.async_copy(cols2d.at[pl.ds(cbase + g * K, K)],
                         colv.at[rset], isems[rset])

    def wait_idx(rset):
        pltpu.make_async_copy(rows2d.at[pl.ds(0, K)], rowv.at[rset],
                              isems[rset]).wait()
        pltpu.make_async_copy(cols2d.at[pl.ds(0, K)], colv.at[rset],
                              isems[rset]).wait()

    def drain_scatters(rset):
        for j in range(K):
            pltpu.make_async_copy(ha_hbm.at[pl.ds(0, EB)], gbuf.at[0, j],
                                  ssems[rset]).wait()

    def body(g, rset, p, drain, prefetch):
        if drain:
            drain_scatters((rset + 2) % 4)
        wait_idx(rset)

        @pl.when(c == 0)
        def _():
            for j in range(K):
                pltpu.async_copy(ha_hbm.at[rowv.at[rset, j]],
                                 gbuf.at[p, j], gsem)

        @pl.when(c == 1)
        def _():
            for j in range(K):
                pltpu.async_copy(hb_hbm.at[rowv.at[rset, j]],
                                 gbuf.at[p, j], gsem)

        for j in range(K):
            pltpu.make_async_copy(ha_hbm.at[pl.ds(0, EB)], gbuf.at[0, j],
                                  gsem).wait()
        for j in range(K):
            pltpu.async_copy(gbuf.at[p, j], acc.at[colv.at[rset, j]],
                             ssems[rset], add=True)
        if prefetch:
            issue_idx(g + 1, (rset + 1) % 4)

    issue_idx(0, 0)
    for g0 in range(4):
        body(g0, g0 % 4, g0 % 2, drain=(g0 >= 2), prefetch=True)

    @pl.loop(1, NG // 4 - 1)
    def _(sg):
        for r in range(4):
            body(sg * 4 + r, r, r % 2, drain=True, prefetch=True)

    for g1 in range(NG - 4, NG):
        body(g1, g1 % 4, g1 % 2, drain=True, prefetch=(g1 < NG - 1))
    drain_scatters((NG - 2) % 4)
    drain_scatters((NG - 1) % 4)

    plsc.subcore_barrier()
    pltpu.sync_copy(acc.at[pl.ds(zbase, ROWS_PER_TILE)],
                    s_hbm.at[pl.ds(c * NPAD + zbase, ROWS_PER_TILE)])


_segsum_call = pl.kernel(
    _segsum_body,
    out_type=jax.ShapeDtypeStruct((NC * NPAD, HALF), jnp.float32),
    mesh=_mesh(),
    compiler_params=_params(),
    scratch_types=[
        pltpu.VMEM((4, K, EB), jnp.int32),
        pltpu.VMEM((4, K, EB), jnp.int32),
        pltpu.VMEM((2, K, EB, HALF), jnp.float32),
        pltpu.SemaphoreType.DMA,
        pltpu.SemaphoreType.DMA,
        pltpu.SemaphoreType.DMA,
        pltpu.SemaphoreType.DMA,
        pltpu.SemaphoreType.DMA,
        pltpu.SemaphoreType.DMA,
        pltpu.SemaphoreType.DMA,
        pltpu.SemaphoreType.DMA,
        pltpu.SemaphoreType.DMA,
        pltpu.VMEM_SHARED((NACC, HALF), jnp.float32),
    ],
)


def _scale0_body(x0_ref, dega_ref, degb_ref, ha_ref, hb_ref):
    deg = dega_ref[...] + degb_ref[...]
    dis = jnp.where(deg > 0, lax.rsqrt(deg), 0.0)[:, :1]
    ha_ref[...] = x0_ref[:, :HALF] * dis
    hb_ref[...] = x0_ref[:, HALF:] * dis


def _scale1_body(s1a_ref, s1b_ref, dega_ref, degb_ref, ha_ref, hb_ref):
    deg = dega_ref[...] + degb_ref[...]
    inv = jnp.where(deg > 0, 1.0 / deg, 0.0)[:, :1]
    ha_ref[...] = s1a_ref[...] * inv
    hb_ref[...] = s1b_ref[...] * inv


def _final_body(x0_ref, s1a_ref, s1b_ref, s2a_ref, s2b_ref,
                dega_ref, degb_ref, z_ref):
    deg = dega_ref[...] + degb_ref[...]
    dis = jnp.where(deg > 0, lax.rsqrt(deg), 0.0)[:, :1]
    sa = (s1a_ref[...] + s2a_ref[...]) * dis
    sb = (s1b_ref[...] + s2b_ref[...]) * dis
    z_ref[...] = (x0_ref[...] + jnp.concatenate([sa, sb], axis=1)) * (1.0 / 3.0)


_RS = 1024
_NB = NPAD // _RS
_half_out = (jax.ShapeDtypeStruct((NPAD, HALF), jnp.float32),) * 2
_scale0_call = pl.pallas_call(
    _scale0_body,
    grid=(_NB,),
    in_specs=[
        pl.BlockSpec((_RS, D), lambda i: (i, 0)),
        pl.BlockSpec((_RS, 8), lambda i: (i, 0)),
        pl.BlockSpec((_RS, 8), lambda i: (_NB + i, 0)),
    ],
    out_specs=[pl.BlockSpec((_RS, HALF), lambda i: (i, 0))] * 2,
    out_shape=_half_out,
)
_scale1_call = pl.pallas_call(
    _scale1_body,
    grid=(_NB,),
    in_specs=[
        pl.BlockSpec((_RS, HALF), lambda i: (i, 0)),
        pl.BlockSpec((_RS, HALF), lambda i: (_NB + i, 0)),
        pl.BlockSpec((_RS, 8), lambda i: (i, 0)),
        pl.BlockSpec((_RS, 8), lambda i: (_NB + i, 0)),
    ],
    out_specs=[pl.BlockSpec((_RS, HALF), lambda i: (i, 0))] * 2,
    out_shape=_half_out,
)
_RF = 400
_NF = NPAD // _RF
_final_call = pl.pallas_call(
    _final_body,
    grid=(N // _RF,),
    in_specs=[
        pl.BlockSpec((_RF, D), lambda i: (i, 0)),
        pl.BlockSpec((_RF, HALF), lambda i: (i, 0)),
        pl.BlockSpec((_RF, HALF), lambda i: (_NF + i, 0)),
        pl.BlockSpec((_RF, HALF), lambda i: (i, 0)),
        pl.BlockSpec((_RF, HALF), lambda i: (_NF + i, 0)),
        pl.BlockSpec((_RF, 8), lambda i: (i, 0)),
        pl.BlockSpec((_RF, 8), lambda i: (_NF + i, 0)),
    ],
    out_specs=pl.BlockSpec((_RF, D), lambda i: (i, 0)),
    out_shape=jax.ShapeDtypeStruct((N, D), jnp.float32),
)


def kernel(emb_table, edge_index, n_id):
    row = edge_index[0].astype(jnp.int32)
    col = edge_index[1].astype(jnp.int32)
    rows2d = jnp.concatenate([row, jnp.zeros((EPAD - E,), jnp.int32)]
                             ).reshape(ECHUNKS, EB)
    cols2d = jnp.concatenate([col, jnp.full((EPAD - E,), N, jnp.int32)]
                             ).reshape(ECHUNKS, EB)
    nidp = jnp.concatenate([n_id.astype(jnp.int32),
                            jnp.zeros((NACC - N,), jnp.int32)])
    zeros8 = jnp.zeros((ROWS_PER_TILE, 8), jnp.float32)
    ones8 = jnp.ones((EB, 8), jnp.float32)
    zeros32 = jnp.zeros((ROWS_PER_TILE, HALF), jnp.float32)

    deg = _deg_call(cols2d, ones8, zeros8)
    x0 = _gather_call(emb_table, nidp)
    h0a, h0b = _scale0_call(x0, deg, deg)
    s1 = _segsum_call(h0a, h0b, rows2d, cols2d, zeros32)
    h1a, h1b = _scale1_call(s1, s1, deg, deg)
    s2 = _segsum_call(h1a, h1b, rows2d, cols2d, zeros32)
    return _final_call(x0, s1, s1, s2, s2, deg, deg)


# speedup vs baseline: 18.6875x; 1.2454x over previous
import jax
import jax.numpy as jnp
from jax import lax
from jax.experimental import pallas as pl
from jax.experimental.pallas import tpu as pltpu
from jax.experimental.pallas import tpu_sc as plsc

N = 50000
E = 800000
D = 64
HALF = 32
NC = 2
NS = 16
NPAD = 51200
NACC = 50176
EB = 400
ECH = E // EB
ROWS_PER_TILE = NACC // NS
NG = ECH // NS
DEG_CH = 62
DEG_TAIL = ECH // NC - NS * DEG_CH
GATHER_ROWS = NACC // (NC * NS)
GEB = 128

_mesh = lambda: plsc.VectorSubcoreMesh(core_axis_name="c", subcore_axis_name="s")
_params = lambda: pltpu.CompilerParams(use_tc_tiling_on_sc=False)


def _deg_body(e3, ones_hbm, zeros_hbm, deg_hbm, colv2, tailv, onesv, dsem, acc):
    c = lax.axis_index("c")
    s = lax.axis_index("s")
    zbase = s * ROWS_PER_TILE
    pltpu.sync_copy(zeros_hbm, acc.at[pl.ds(zbase, ROWS_PER_TILE)])
    pltpu.sync_copy(ones_hbm, onesv)
    cbase = c * (ECH // NC) + s * DEG_CH
    pltpu.sync_copy(e3.at[pl.ds(cbase, DEG_CH)], colv2)
    plsc.subcore_barrier()

    @pl.loop(0, DEG_CH // 2)
    def _(i):
        d0 = pltpu.async_copy(onesv, acc.at[colv2.at[2 * i, 1]], dsem,
                              add=True)
        d1 = pltpu.async_copy(onesv, acc.at[colv2.at[2 * i + 1, 1]], dsem,
                              add=True)
        d0.wait()
        d1.wait()

    @pl.when(s < DEG_TAIL)
    def _():
        cidx = c * (ECH // NC) + NS * DEG_CH + s
        pltpu.sync_copy(e3.at[pl.ds(cidx, 1)], tailv)
        pltpu.sync_copy(onesv, acc.at[tailv.at[0, 1]], add=True)

    plsc.subcore_barrier()
    pltpu.sync_copy(acc.at[pl.ds(zbase, ROWS_PER_TILE)],
                    deg_hbm.at[pl.ds(c * NPAD + zbase, ROWS_PER_TILE)])


_deg_call = pl.kernel(
    _deg_body,
    out_type=jax.ShapeDtypeStruct((NC * NPAD, 8), jnp.float32),
    mesh=_mesh(),
    compiler_params=_params(),
    scratch_types=[
        pltpu.VMEM((DEG_CH, 2, EB), jnp.int32),
        pltpu.VMEM((1, 2, EB), jnp.int32),
        pltpu.VMEM((EB, 8), jnp.float32),
        pltpu.SemaphoreType.DMA,
        pltpu.VMEM_SHARED((NACC, 8), jnp.float32),
    ],
)


def _gather_body(emb_hbm, nid_hbm, x0_hbm, idxv, gbuf, sem):
    c = lax.axis_index("c")
    s = lax.axis_index("s")
    off = (s * NC + c) * GATHER_ROWS
    pltpu.sync_copy(nid_hbm.at[pl.ds(off, GATHER_ROWS)], idxv)
    descs = []
    for j in range(GATHER_ROWS // GEB):
        descs.append(pltpu.async_copy(
            emb_hbm.at[idxv.at[pl.ds(j * GEB, GEB)]],
            gbuf.at[pl.ds(j * GEB, GEB)], sem))
    rem = GATHER_ROWS % GEB
    if rem:
        base = (GATHER_ROWS // GEB) * GEB
        descs.append(pltpu.async_copy(
            emb_hbm.at[idxv.at[pl.ds(base, rem)]],
            gbuf.at[pl.ds(base, rem)], sem))
    for d in descs:
        d.wait()
    pltpu.sync_copy(gbuf, x0_hbm.at[pl.ds(off, GATHER_ROWS)])


_gather_call = pl.kernel(
    _gather_body,
    out_type=jax.ShapeDtypeStruct((NPAD, D), jnp.float32),
    mesh=_mesh(),
    compiler_params=_params(),
    scratch_types=[
        pltpu.VMEM((GATHER_ROWS,), jnp.int32),
        pltpu.VMEM((GATHER_ROWS, D), jnp.float32),
        pltpu.SemaphoreType.DMA,
    ],
)


def _segsum_body(ha_hbm, hb_hbm, e3, zeros_hbm, s_hbm,
                 idx4, gbuf,
                 isem0, isem1, isem2, isem3, gsem, ssem0, ssem1, ssem2, ssem3,
                 acc):
    c = lax.axis_index("c")
    s = lax.axis_index("s")
    isems = (isem0, isem1, isem2, isem3)
    ssems = (ssem0, ssem1, ssem2, ssem3)
    zbase = s * ROWS_PER_TILE
    cbase = s * NG

    pltpu.sync_copy(zeros_hbm, acc.at[pl.ds(zbase, ROWS_PER_TILE)])
    plsc.subcore_barrier()

    def issue_idx(g, rset):
        pltpu.async_copy(e3.at[pl.ds(cbase + g, 1)], idx4.at[pl.ds(rset, 1)],
                         isems[rset])

    def wait_idx(rset):
        pltpu.make_async_copy(e3.at[pl.ds(0, 1)], idx4.at[pl.ds(rset, 1)],
                              isems[rset]).wait()

    def drain_scatter(rset):
        pltpu.make_async_copy(ha_hbm.at[pl.ds(0, EB)], gbuf.at[0],
                              ssems[rset]).wait()

    def body(g, rset, p, drain, prefetch):
        if drain:
            drain_scatter((rset + 2) % 4)
        wait_idx(rset)

        @pl.when(c == 0)
        def _():
            pltpu.async_copy(ha_hbm.at[idx4.at[rset, 0]], gbuf.at[p], gsem)

        @pl.when(c == 1)
        def _():
            pltpu.async_copy(hb_hbm.at[idx4.at[rset, 0]], gbuf.at[p], gsem)

        pltpu.make_async_copy(ha_hbm.at[pl.ds(0, EB)], gbuf.at[0],
                              gsem).wait()
        pltpu.async_copy(gbuf.at[p], acc.at[idx4.at[rset, 1]],
                         ssems[rset], add=True)
        if prefetch:
            issue_idx(g + 1, (rset + 1) % 4)

    issue_idx(0, 0)
    for g0 in range(4):
        body(g0, g0 % 4, g0 % 2, drain=(g0 >= 2), prefetch=True)

    @pl.loop(1, 30)
    def _(sg):
        for r in range(4):
            body(sg * 4 + r, r, r % 2, drain=True, prefetch=True)

    for g1 in range(120, NG):
        body(g1, g1 % 4, g1 % 2, drain=True, prefetch=(g1 < NG - 1))
    drain_scatter((NG - 2) % 4)
    drain_scatter((NG - 1) % 4)

    plsc.subcore_barrier()
    pltpu.sync_copy(acc.at[pl.ds(zbase, ROWS_PER_TILE)],
                    s_hbm.at[pl.ds(c * NPAD + zbase, ROWS_PER_TILE)])


_segsum_call = pl.kernel(
    _segsum_body,
    out_type=jax.ShapeDtypeStruct((NC * NPAD, HALF), jnp.float32),
    mesh=_mesh(),
    compiler_params=_params(),
    scratch_types=[
        pltpu.VMEM((4, 2, EB), jnp.int32),
        pltpu.VMEM((2, EB, HALF), jnp.float32),
        pltpu.SemaphoreType.DMA,
        pltpu.SemaphoreType.DMA,
        pltpu.SemaphoreType.DMA,
        pltpu.SemaphoreType.DMA,
        pltpu.SemaphoreType.DMA,
        pltpu.SemaphoreType.DMA,
        pltpu.SemaphoreType.DMA,
        pltpu.SemaphoreType.DMA,
        pltpu.SemaphoreType.DMA,
        pltpu.VMEM_SHARED((NACC, HALF), jnp.float32),
    ],
)


def _scale0_body(x0_ref, dega_ref, degb_ref, ha_ref, hb_ref):
    deg = dega_ref[...] + degb_ref[...]
    dis = jnp.where(deg > 0, lax.rsqrt(deg), 0.0)[:, :1]
    ha_ref[...] = x0_ref[:, :HALF] * dis
    hb_ref[...] = x0_ref[:, HALF:] * dis


def _scale1_body(s1a_ref, s1b_ref, dega_ref, degb_ref, ha_ref, hb_ref):
    deg = dega_ref[...] + degb_ref[...]
    inv = jnp.where(deg > 0, 1.0 / deg, 0.0)[:, :1]
    ha_ref[...] = s1a_ref[...] * inv
    hb_ref[...] = s1b_ref[...] * inv


def _final_body(x0_ref, s1a_ref, s1b_ref, s2a_ref, s2b_ref,
                dega_ref, degb_ref, z_ref):
    deg = dega_ref[...] + degb_ref[...]
    dis = jnp.where(deg > 0, lax.rsqrt(deg), 0.0)[:, :1]
    sa = (s1a_ref[...] + s2a_ref[...]) * dis
    sb = (s1b_ref[...] + s2b_ref[...]) * dis
    z_ref[...] = (x0_ref[...] + jnp.concatenate([sa, sb], axis=1)) * (1.0 / 3.0)


_RS = 1024
_NB = NPAD // _RS
_half_out = (jax.ShapeDtypeStruct((NPAD, HALF), jnp.float32),) * 2
_scale0_call = pl.pallas_call(
    _scale0_body,
    grid=(_NB,),
    in_specs=[
        pl.BlockSpec((_RS, D), lambda i: (i, 0)),
        pl.BlockSpec((_RS, 8), lambda i: (i, 0)),
        pl.BlockSpec((_RS, 8), lambda i: (_NB + i, 0)),
    ],
    out_specs=[pl.BlockSpec((_RS, HALF), lambda i: (i, 0))] * 2,
    out_shape=_half_out,
)
_scale1_call = pl.pallas_call(
    _scale1_body,
    grid=(_NB,),
    in_specs=[
        pl.BlockSpec((_RS, HALF), lambda i: (i, 0)),
        pl.BlockSpec((_RS, HALF), lambda i: (_NB + i, 0)),
        pl.BlockSpec((_RS, 8), lambda i: (i, 0)),
        pl.BlockSpec((_RS, 8), lambda i: (_NB + i, 0)),
    ],
    out_specs=[pl.BlockSpec((_RS, HALF), lambda i: (i, 0))] * 2,
    out_shape=_half_out,
)
_RF = 400
_NF = NPAD // _RF
_final_call = pl.pallas_call(
    _final_body,
    grid=(N // _RF,),
    in_specs=[
        pl.BlockSpec((_RF, D), lambda i: (i, 0)),
        pl.BlockSpec((_RF, HALF), lambda i: (i, 0)),
        pl.BlockSpec((_RF, HALF), lambda i: (_NF + i, 0)),
        pl.BlockSpec((_RF, HALF), lambda i: (i, 0)),
        pl.BlockSpec((_RF, HALF), lambda i: (_NF + i, 0)),
        pl.BlockSpec((_RF, 8), lambda i: (i, 0)),
        pl.BlockSpec((_RF, 8), lambda i: (_NF + i, 0)),
    ],
    out_specs=pl.BlockSpec((_RF, D), lambda i: (i, 0)),
    out_shape=jax.ShapeDtypeStruct((N, D), jnp.float32),
)


def kernel(emb_table, edge_index, n_id):
    e3 = edge_index.astype(jnp.int32).reshape(2, ECH, EB).transpose(1, 0, 2)
    nidp = jnp.concatenate([n_id.astype(jnp.int32),
                            jnp.zeros((NACC - N,), jnp.int32)])
    zeros8 = jnp.zeros((ROWS_PER_TILE, 8), jnp.float32)
    ones8 = jnp.ones((EB, 8), jnp.float32)
    zeros32 = jnp.zeros((ROWS_PER_TILE, HALF), jnp.float32)

    deg = _deg_call(e3, ones8, zeros8)
    x0 = _gather_call(emb_table, nidp)
    h0a, h0b = _scale0_call(x0, deg, deg)
    s1 = _segsum_call(h0a, h0b, e3, zeros32)
    h1a, h1b = _scale1_call(s1, s1, deg, deg)
    s2 = _segsum_call(h1a, h1b, e3, zeros32)
    return _final_call(x0, s1, s1, s2, s2, deg, deg)


# speedup vs baseline: 20.9517x; 1.1212x over previous
import jax
import jax.numpy as jnp
from jax import lax
from jax.experimental import pallas as pl
from jax.experimental.pallas import tpu as pltpu
from jax.experimental.pallas import tpu_sc as plsc

N = 50000
E = 800000
D = 64
HALF = 32
NC = 2
NS = 16
NPAD = 51200
NACC = 50176
EB = 400
ECH = E // EB
ROWS_PER_TILE = NACC // NS
NG = ECH // NS
DEG_CH = 62
DEG_TAIL = ECH // NC - NS * DEG_CH
GATHER_ROWS = NACC // (NC * NS)
GCH = 224
W = 112
MAGIC = 0x5F3759DF

_mesh = lambda: plsc.VectorSubcoreMesh(core_axis_name="c", subcore_axis_name="s")
_params = lambda: pltpu.CompilerParams(use_tc_tiling_on_sc=False, needs_layout_passes=False)


def _rsqrt16(dg):
    i = plsc.bitcast(dg, jnp.int32)
    y = plsc.bitcast(jnp.int32(MAGIC) - (i >> 1), jnp.float32)
    for _ in range(3):
        y = y * (1.5 - 0.5 * dg * y * y)
    return jnp.where(dg > 0, y, 0.0)


def _deg_body(e3, ones_hbm, zeros_hbm, deg_hbm, colv2, tailv, onesv, dsem, acc):
    c = lax.axis_index("c")
    s = lax.axis_index("s")
    zbase = s * ROWS_PER_TILE
    pltpu.sync_copy(zeros_hbm, acc.at[pl.ds(zbase, ROWS_PER_TILE)])
    pltpu.sync_copy(ones_hbm, onesv)
    cbase = c * (ECH // NC) + s * DEG_CH
    pltpu.sync_copy(e3.at[pl.ds(cbase, DEG_CH)], colv2)
    plsc.subcore_barrier()

    @pl.loop(0, DEG_CH // 2)
    def _(i):
        d0 = pltpu.async_copy(onesv, acc.at[colv2.at[2 * i, 1]], dsem,
                              add=True)
        d1 = pltpu.async_copy(onesv, acc.at[colv2.at[2 * i + 1, 1]], dsem,
                              add=True)
        d0.wait()
        d1.wait()

    @pl.when(s < DEG_TAIL)
    def _():
        cidx = c * (ECH // NC) + NS * DEG_CH + s
        pltpu.sync_copy(e3.at[pl.ds(cidx, 1)], tailv)
        pltpu.sync_copy(onesv, acc.at[tailv.at[0, 1]], add=True)

    plsc.subcore_barrier()
    pltpu.sync_copy(acc.at[pl.ds(zbase, ROWS_PER_TILE)],
                    deg_hbm.at[pl.ds(c * NPAD + zbase, ROWS_PER_TILE)])


_deg_call = pl.kernel(
    _deg_body,
    out_type=jax.ShapeDtypeStruct((NC * NPAD, 16), jnp.float32),
    mesh=_mesh(),
    compiler_params=_params(),
    scratch_types=[
        pltpu.VMEM((DEG_CH, 2, EB), jnp.int32),
        pltpu.VMEM((1, 2, EB), jnp.int32),
        pltpu.VMEM((EB, 16), jnp.float32),
        pltpu.SemaphoreType.DMA,
        pltpu.VMEM_SHARED((NACC, 16), jnp.float32),
    ],
)


def _gs_body(emb_hbm, nid_hbm, degp_hbm,
             x0a_hbm, x0b_hbm, h0a_hbm, h0b_hbm, dis_hbm,
             idxv, gbuf, dav, dbv, disv, x0av, x0bv, hav, hbv, sem):
    c = lax.axis_index("c")
    s = lax.axis_index("s")
    off = (s * NC + c) * GATHER_ROWS
    pltpu.sync_copy(nid_hbm.at[pl.ds(off, GATHER_ROWS)], idxv)
    for ch in range(GATHER_ROWS // GCH):
        rb = off + ch * GCH
        pltpu.async_copy(emb_hbm.at[idxv.at[pl.ds(ch * GCH, GCH)]],
                         gbuf, sem).wait()
        pltpu.sync_copy(degp_hbm.at[pl.ds(rb, GCH)], dav)
        pltpu.sync_copy(degp_hbm.at[pl.ds(NPAD + rb, GCH)], dbv)

        @pl.loop(0, GCH)
        def _(r):
            dg = dav[r, pl.ds(0, 16)] + dbv[r, pl.ds(0, 16)]
            y = _rsqrt16(dg)
            disv[r, pl.ds(0, 16)] = y
            disv[r, pl.ds(16, 16)] = y
            g0 = gbuf[r, pl.ds(0, 16)]
            g1 = gbuf[r, pl.ds(16, 16)]
            g2 = gbuf[r, pl.ds(32, 16)]
            g3 = gbuf[r, pl.ds(48, 16)]
            x0av[r, pl.ds(0, 16)] = g0
            x0av[r, pl.ds(16, 16)] = g1
            x0bv[r, pl.ds(0, 16)] = g2
            x0bv[r, pl.ds(16, 16)] = g3
            hav[r, pl.ds(0, 16)] = g0 * y
            hav[r, pl.ds(16, 16)] = g1 * y
            hbv[r, pl.ds(0, 16)] = g2 * y
            hbv[r, pl.ds(16, 16)] = g3 * y

        pltpu.sync_copy(x0av, x0a_hbm.at[pl.ds(rb, GCH)])
        pltpu.sync_copy(x0bv, x0b_hbm.at[pl.ds(rb, GCH)])
        pltpu.sync_copy(hav, h0a_hbm.at[pl.ds(rb, GCH)])
        pltpu.sync_copy(hbv, h0b_hbm.at[pl.ds(rb, GCH)])
        pltpu.sync_copy(disv, dis_hbm.at[pl.ds(rb, GCH)])


_gs_call = pl.kernel(
    _gs_body,
    out_type=(jax.ShapeDtypeStruct((NPAD, HALF), jnp.float32),) * 4
    + (jax.ShapeDtypeStruct((NPAD, 2 * 16), jnp.float32),),
    mesh=_mesh(),
    compiler_params=_params(),
    scratch_types=[
        pltpu.VMEM((GATHER_ROWS,), jnp.int32),
        pltpu.VMEM((GCH, D), jnp.float32),
        pltpu.VMEM((GCH, 16), jnp.float32),
        pltpu.VMEM((GCH, 16), jnp.float32),
        pltpu.VMEM((GCH, 32), jnp.float32),
        pltpu.VMEM((GCH, HALF), jnp.float32),
        pltpu.VMEM((GCH, HALF), jnp.float32),
        pltpu.VMEM((GCH, HALF), jnp.float32),
        pltpu.VMEM((GCH, HALF), jnp.float32),
        pltpu.SemaphoreType.DMA,
    ],
)


def _make_segsum(last):
    def body(ha_hbm, hb_hbm, e3, zeros_hbm, dis_hbm, *rest):
        if last:
            x0a_hbm, x0b_hbm, x1_hbm, z_hbm = rest[:4]
            rest = rest[4:]
        else:
            x_hbm, h1a_hbm, h1b_hbm = rest[:3]
            rest = rest[3:]
        (idx4, gbuf, isem0, isem1, isem2, isem3, gsem,
         ssem0, ssem1, ssem2, ssem3, acc) = rest
        c = lax.axis_index("c")
        s = lax.axis_index("s")
        isems = (isem0, isem1, isem2, isem3)
        ssems = (ssem0, ssem1, ssem2, ssem3)
        zbase = s * ROWS_PER_TILE
        cbase = s * NG

        pltpu.sync_copy(zeros_hbm, acc.at[pl.ds(zbase, ROWS_PER_TILE)])
        plsc.subcore_barrier()

        def issue_idx(g, rset):
            pltpu.async_copy(e3.at[pl.ds(cbase + g, 1)],
                             idx4.at[pl.ds(rset, 1)], isems[rset])

        def wait_idx(rset):
            pltpu.make_async_copy(e3.at[pl.ds(0, 1)],
                                  idx4.at[pl.ds(rset, 1)], isems[rset]).wait()

        def drain_scatter(rset):
            pltpu.make_async_copy(ha_hbm.at[pl.ds(0, EB)],
                                  gbuf.at[pl.ds(0, EB)], ssems[rset]).wait()

        def sbody(g, rset, p, drain, prefetch):
            if drain:
                drain_scatter((rset + 2) % 4)
            wait_idx(rset)

            @pl.when(c == 0)
            def _():
                pltpu.async_copy(ha_hbm.at[idx4.at[rset, 0]],
                                 gbuf.at[pl.ds(p * EB, EB)], gsem)

            @pl.when(c == 1)
            def _():
                pltpu.async_copy(hb_hbm.at[idx4.at[rset, 0]],
                                 gbuf.at[pl.ds(p * EB, EB)], gsem)

            pltpu.make_async_copy(ha_hbm.at[pl.ds(0, EB)],
                                  gbuf.at[pl.ds(0, EB)], gsem).wait()
            pltpu.async_copy(gbuf.at[pl.ds(p * EB, EB)],
                             acc.at[idx4.at[rset, 1]], ssems[rset], add=True)
            if prefetch:
                issue_idx(g + 1, (rset + 1) % 4)

        issue_idx(0, 0)
        for g0 in range(4):
            sbody(g0, g0 % 4, g0 % 2, drain=(g0 >= 2), prefetch=True)

        @pl.loop(1, 30)
        def _(sg):
            for r in range(4):
                sbody(sg * 4 + r, r, r % 2, drain=True, prefetch=True)

        for g1 in range(120, NG):
            sbody(g1, g1 % 4, g1 % 2, drain=True, prefetch=(g1 < NG - 1))
        drain_scatter((NG - 2) % 4)
        drain_scatter((NG - 1) % 4)

        plsc.subcore_barrier()

        @pl.loop(0, ROWS_PER_TILE // W)
        def _(wch):
            rb = zbase + wch * W
            pltpu.sync_copy(acc.at[pl.ds(rb, W)], gbuf.at[pl.ds(0, W)])
            pltpu.sync_copy(dis_hbm.at[pl.ds(rb, W)], gbuf.at[pl.ds(W, W)])
            if last:
                pltpu.sync_copy(x1_hbm.at[pl.ds(c * NPAD + rb, W)],
                                gbuf.at[pl.ds(2 * W, W)])

                @pl.when(c == 0)
                def _():
                    pltpu.sync_copy(x0a_hbm.at[pl.ds(rb, W)],
                                    gbuf.at[pl.ds(3 * W, W)])

                @pl.when(c == 1)
                def _():
                    pltpu.sync_copy(x0b_hbm.at[pl.ds(rb, W)],
                                    gbuf.at[pl.ds(3 * W, W)])

                @pl.loop(0, W)
                def _(r):
                    dv = gbuf[W + r, pl.ds(0, 16)]
                    for q in (0, 16):
                        a = gbuf[r, pl.ds(q, 16)] * dv
                        gbuf[4 * W + r, pl.ds(q, 16)] = (
                            gbuf[3 * W + r, pl.ds(q, 16)]
                            + gbuf[2 * W + r, pl.ds(q, 16)] + a) * (1.0 / 3.0)

                pltpu.sync_copy(gbuf.at[pl.ds(4 * W, W)],
                                z_hbm.at[pl.ds(c * NPAD + rb, W)])
            else:
                @pl.loop(0, W)
                def _(r):
                    dv = gbuf[W + r, pl.ds(0, 16)]
                    d2 = dv * dv
                    for q in (0, 16):
                        sv = gbuf[r, pl.ds(q, 16)]
                        gbuf[2 * W + r, pl.ds(q, 16)] = sv * dv
                        gbuf[3 * W + r, pl.ds(q, 16)] = sv * d2

                pltpu.sync_copy(gbuf.at[pl.ds(2 * W, W)],
                                x_hbm.at[pl.ds(c * NPAD + rb, W)])

                @pl.when(c == 0)
                def _():
                    pltpu.sync_copy(gbuf.at[pl.ds(3 * W, W)],
                                    h1a_hbm.at[pl.ds(rb, W)])

                @pl.when(c == 1)
                def _():
                    pltpu.sync_copy(gbuf.at[pl.ds(3 * W, W)],
                                    h1b_hbm.at[pl.ds(rb, W)])

    if last:
        outs = jax.ShapeDtypeStruct((NC * NPAD, HALF), jnp.float32)
    else:
        outs = (jax.ShapeDtypeStruct((NC * NPAD, HALF), jnp.float32),
                jax.ShapeDtypeStruct((NPAD, HALF), jnp.float32),
                jax.ShapeDtypeStruct((NPAD, HALF), jnp.float32))
    return pl.kernel(
        body,
        out_type=outs,
        mesh=_mesh(),
        compiler_params=_params(),
        scratch_types=[
            pltpu.VMEM((4, 2, EB), jnp.int32),
            pltpu.VMEM((2 * EB, HALF), jnp.float32),
            pltpu.SemaphoreType.DMA,
            pltpu.SemaphoreType.DMA,
            pltpu.SemaphoreType.DMA,
            pltpu.SemaphoreType.DMA,
            pltpu.SemaphoreType.DMA,
            pltpu.SemaphoreType.DMA,
            pltpu.SemaphoreType.DMA,
            pltpu.SemaphoreType.DMA,
            pltpu.SemaphoreType.DMA,
            pltpu.VMEM_SHARED((NACC, HALF), jnp.float32),
        ],
    )


_seg_mid = _make_segsum(last=False)
_seg_last = _make_segsum(last=True)


def _z_body(za_ref, zb_ref, z_ref):
    z_ref[...] = jnp.concatenate([za_ref[...], zb_ref[...]], axis=1)


_ZR = 400
_z_call = pl.pallas_call(
    _z_body,
    grid=(N // _ZR,),
    in_specs=[
        pl.BlockSpec((_ZR, HALF), lambda i: (i, 0)),
        pl.BlockSpec((_ZR, HALF), lambda i: (NPAD // _ZR + i, 0)),
    ],
    out_specs=pl.BlockSpec((_ZR, D), lambda i: (i, 0)),
    out_shape=jax.ShapeDtypeStruct((N, D), jnp.float32),
)


def kernel(emb_table, edge_index, n_id):
    e3 = edge_index.astype(jnp.int32).reshape(2, ECH, EB).transpose(1, 0, 2)
    nidp = jnp.concatenate([n_id.astype(jnp.int32),
                            jnp.zeros((NACC - N,), jnp.int32)])
    ones16 = jnp.ones((EB, 16), jnp.float32)
    zeros16 = jnp.zeros((ROWS_PER_TILE, 16), jnp.float32)
    zeros32 = jnp.zeros((ROWS_PER_TILE, HALF), jnp.float32)

    degp = _deg_call(e3, ones16, zeros16)
    x0a, x0b, h0a, h0b, dis32 = _gs_call(emb_table, nidp, degp)
    x1, h1a, h1b = _seg_mid(h0a, h0b, e3, zeros32, dis32)
    z2 = _seg_last(h1a, h1b, e3, zeros32, dis32, x0a, x0b, x1)
    return _z_call(z2, z2)


# speedup vs baseline: 23.9496x; 1.1431x over previous
import jax
import jax.numpy as jnp
from jax import lax
from jax.experimental import pallas as pl
from jax.experimental.pallas import tpu as pltpu
from jax.experimental.pallas import tpu_sc as plsc

N = 50000
E = 800000
D = 64
HALF = 32
NC = 2
NS = 16
NPAD = 51200
NACC = 50176
EB = 400
ECH = E // EB
ROWS_PER_TILE = NACC // NS
NG = ECH // NS
DEG_CH = 62
DEG_TAIL = ECH // NC - NS * DEG_CH
GATHER_ROWS = NACC // (NC * NS)
GCH = 224
W = 125
WROWS = N // NS
NWCH = WROWS // W
MAGIC = 0x5F3759DF

_mesh = lambda: plsc.VectorSubcoreMesh(core_axis_name="c", subcore_axis_name="s")
_params = lambda: pltpu.CompilerParams(use_tc_tiling_on_sc=False, needs_layout_passes=False)


def _rsqrt16(dg):
    i = plsc.bitcast(dg, jnp.int32)
    y = plsc.bitcast(jnp.int32(MAGIC) - (i >> 1), jnp.float32)
    for _ in range(3):
        y = y * (1.5 - 0.5 * dg * y * y)
    return jnp.where(dg > 0, y, 0.0)


def _deg_body(e3, ones_hbm, zeros_hbm, deg_hbm, colv2, tailv, onesv, dsem, acc):
    c = lax.axis_index("c")
    s = lax.axis_index("s")
    zbase = s * ROWS_PER_TILE
    pltpu.sync_copy(zeros_hbm, acc.at[pl.ds(zbase, ROWS_PER_TILE)])
    pltpu.sync_copy(ones_hbm, onesv)
    cbase = c * (ECH // NC) + s * DEG_CH
    pltpu.sync_copy(e3.at[pl.ds(cbase, DEG_CH)], colv2)
    plsc.subcore_barrier()

    @pl.loop(0, DEG_CH // 2)
    def _(i):
        d0 = pltpu.async_copy(onesv, acc.at[colv2.at[2 * i, 1]], dsem,
                              add=True)
        d1 = pltpu.async_copy(onesv, acc.at[colv2.at[2 * i + 1, 1]], dsem,
                              add=True)
        d0.wait()
        d1.wait()

    @pl.when(s < DEG_TAIL)
    def _():
        cidx = c * (ECH // NC) + NS * DEG_CH + s
        pltpu.sync_copy(e3.at[pl.ds(cidx, 1)], tailv)
        pltpu.sync_copy(onesv, acc.at[tailv.at[0, 1]], add=True)

    plsc.subcore_barrier()
    pltpu.sync_copy(acc.at[pl.ds(zbase, ROWS_PER_TILE)],
                    deg_hbm.at[pl.ds(c * NPAD + zbase, ROWS_PER_TILE)])


_deg_call = pl.kernel(
    _deg_body,
    out_type=jax.ShapeDtypeStruct((NC * NPAD, 16), jnp.float32),
    mesh=_mesh(),
    compiler_params=_params(),
    scratch_types=[
        pltpu.VMEM((DEG_CH, 2, EB), jnp.int32),
        pltpu.VMEM((1, 2, EB), jnp.int32),
        pltpu.VMEM((EB, 16), jnp.float32),
        pltpu.SemaphoreType.DMA,
        pltpu.VMEM_SHARED((NACC, 16), jnp.float32),
    ],
)


def _gs_body(emb_hbm, nid_hbm, degp_hbm,
             x0_hbm, h0a_hbm, h0b_hbm, dis_hbm,
             idxv, gbuf, dav, dbv, disv, hav, hbv, sem):
    c = lax.axis_index("c")
    s = lax.axis_index("s")
    off = (s * NC + c) * GATHER_ROWS
    pltpu.sync_copy(nid_hbm.at[pl.ds(off, GATHER_ROWS)], idxv)
    for ch in range(GATHER_ROWS // GCH):
        rb = off + ch * GCH
        pltpu.async_copy(emb_hbm.at[idxv.at[pl.ds(ch * GCH, GCH)]],
                         gbuf, sem).wait()
        pltpu.sync_copy(degp_hbm.at[pl.ds(rb, GCH)], dav)
        pltpu.sync_copy(degp_hbm.at[pl.ds(NPAD + rb, GCH)], dbv)

        @pl.loop(0, GCH)
        def _(r):
            dg = dav[r, pl.ds(0, 16)] + dbv[r, pl.ds(0, 16)]
            y = _rsqrt16(dg)
            disv[r, pl.ds(0, 16)] = y
            disv[r, pl.ds(16, 16)] = y
            g0 = gbuf[r, pl.ds(0, 16)]
            g1 = gbuf[r, pl.ds(16, 16)]
            g2 = gbuf[r, pl.ds(32, 16)]
            g3 = gbuf[r, pl.ds(48, 16)]
            hav[r, pl.ds(0, 16)] = g0 * y
            hav[r, pl.ds(16, 16)] = g1 * y
            hbv[r, pl.ds(0, 16)] = g2 * y
            hbv[r, pl.ds(16, 16)] = g3 * y

        pltpu.sync_copy(gbuf, x0_hbm.at[pl.ds(rb, GCH)])
        pltpu.sync_copy(hav, h0a_hbm.at[pl.ds(rb, GCH)])
        pltpu.sync_copy(hbv, h0b_hbm.at[pl.ds(rb, GCH)])
        pltpu.sync_copy(disv, dis_hbm.at[pl.ds(rb, GCH)])


_gs_call = pl.kernel(
    _gs_body,
    out_type=(jax.ShapeDtypeStruct((NPAD, D), jnp.float32),
              jax.ShapeDtypeStruct((NPAD, HALF), jnp.float32),
              jax.ShapeDtypeStruct((NPAD, HALF), jnp.float32),
              jax.ShapeDtypeStruct((NPAD, 2 * 16), jnp.float32)),
    mesh=_mesh(),
    compiler_params=_params(),
    scratch_types=[
        pltpu.VMEM((GATHER_ROWS,), jnp.int32),
        pltpu.VMEM((GCH, D), jnp.float32),
        pltpu.VMEM((GCH, 16), jnp.float32),
        pltpu.VMEM((GCH, 16), jnp.float32),
        pltpu.VMEM((GCH, 32), jnp.float32),
        pltpu.VMEM((GCH, HALF), jnp.float32),
        pltpu.VMEM((GCH, HALF), jnp.float32),
        pltpu.SemaphoreType.DMA,
    ],
)


def _make_segsum(last):
    def body(ha_hbm, hb_hbm, e3, zeros_hbm, dis_hbm, *rest):
        if last:
            x0_hbm, x1_hbm, z_hbm = rest[:3]
            rest = rest[3:]
        else:
            x_hbm, h1a_hbm, h1b_hbm = rest[:3]
            rest = rest[3:]
        (idx4, gbuf, isem0, isem1, isem2, isem3, gsem,
         ssem0, ssem1, ssem2, ssem3, acc) = rest
        c = lax.axis_index("c")
        s = lax.axis_index("s")
        isems = (isem0, isem1, isem2, isem3)
        ssems = (ssem0, ssem1, ssem2, ssem3)
        zbase = s * ROWS_PER_TILE
        cbase = s * NG

        pltpu.sync_copy(zeros_hbm, acc.at[pl.ds(zbase, ROWS_PER_TILE)])
        plsc.subcore_barrier()

        def issue_idx(g, rset):
            pltpu.async_copy(e3.at[pl.ds(cbase + g, 1)],
                             idx4.at[pl.ds(rset, 1)], isems[rset])

        def wait_idx(rset):
            pltpu.make_async_copy(e3.at[pl.ds(0, 1)],
                                  idx4.at[pl.ds(rset, 1)], isems[rset]).wait()

        def drain_scatter(rset):
            pltpu.make_async_copy(ha_hbm.at[pl.ds(0, EB)],
                                  gbuf.at[pl.ds(0, EB)], ssems[rset]).wait()

        def sbody(g, rset, p, drain, prefetch):
            if drain:
                drain_scatter((rset + 2) % 4)
            wait_idx(rset)

            @pl.when(c == 0)
            def _():
                pltpu.async_copy(ha_hbm.at[idx4.at[rset, 0]],
                                 gbuf.at[pl.ds(p * EB, EB)], gsem)

            @pl.when(c == 1)
            def _():
                pltpu.async_copy(hb_hbm.at[idx4.at[rset, 0]],
                                 gbuf.at[pl.ds(p * EB, EB)], gsem)

            pltpu.make_async_copy(ha_hbm.at[pl.ds(0, EB)],
                                  gbuf.at[pl.ds(0, EB)], gsem).wait()
            pltpu.async_copy(gbuf.at[pl.ds(p * EB, EB)],
                             acc.at[idx4.at[rset, 1]], ssems[rset], add=True)
            if prefetch:
                issue_idx(g + 1, (rset + 1) % 4)

        issue_idx(0, 0)
        for g0 in range(4):
            sbody(g0, g0 % 4, g0 % 2, drain=(g0 >= 2), prefetch=True)

        @pl.loop(1, 30)
        def _(sg):
            for r in range(4):
                sbody(sg * 4 + r, r, r % 2, drain=True, prefetch=True)

        for g1 in range(120, NG):
            sbody(g1, g1 % 4, g1 % 2, drain=True, prefetch=(g1 < NG - 1))
        drain_scatter((NG - 2) % 4)
        drain_scatter((NG - 1) % 4)

        plsc.subcore_barrier()

        @pl.loop(0, NWCH)
        def _(wch):
            rb = s * WROWS + wch * W
            pltpu.sync_copy(acc.at[pl.ds(rb, W)], gbuf.at[pl.ds(0, W)])
            pltpu.sync_copy(dis_hbm.at[pl.ds(rb, W)], gbuf.at[pl.ds(W, W)])
            if last:
                pltpu.sync_copy(x1_hbm.at[pl.ds(c * NPAD + rb, W)],
                                gbuf.at[pl.ds(2 * W, W)])

                @pl.when(c == 0)
                def _():
                    pltpu.sync_copy(x0_hbm.at[pl.ds(rb, W), pl.ds(0, HALF)],
                                    gbuf.at[pl.ds(3 * W, W)])

                @pl.when(c == 1)
                def _():
                    pltpu.sync_copy(x0_hbm.at[pl.ds(rb, W), pl.ds(HALF, HALF)],
                                    gbuf.at[pl.ds(3 * W, W)])

                @pl.loop(0, W)
                def _(r):
                    dv = gbuf[W + r, pl.ds(0, 16)]
                    for q in (0, 16):
                        a = gbuf[r, pl.ds(q, 16)] * dv
                        gbuf[4 * W + r, pl.ds(q, 16)] = (
                            gbuf[3 * W + r, pl.ds(q, 16)]
                            + gbuf[2 * W + r, pl.ds(q, 16)] + a) * (1.0 / 3.0)

                @pl.when(c == 0)
                def _():
                    pltpu.sync_copy(gbuf.at[pl.ds(4 * W, W)],
                                    z_hbm.at[pl.ds(rb, W), pl.ds(0, HALF)])

                @pl.when(c == 1)
                def _():
                    pltpu.sync_copy(gbuf.at[pl.ds(4 * W, W)],
                                    z_hbm.at[pl.ds(rb, W), pl.ds(HALF, HALF)])
            else:
                @pl.loop(0, W)
                def _(r):
                    dv = gbuf[W + r, pl.ds(0, 16)]
                    d2 = dv * dv
                    for q in (0, 16):
                        sv = gbuf[r, pl.ds(q, 16)]
                        gbuf[2 * W + r, pl.ds(q, 16)] = sv * dv
                        gbuf[3 * W + r, pl.ds(q, 16)] = sv * d2

                pltpu.sync_copy(gbuf.at[pl.ds(2 * W, W)],
                                x_hbm.at[pl.ds(c * NPAD + rb, W)])

                @pl.when(c == 0)
                def _():
                    pltpu.sync_copy(gbuf.at[pl.ds(3 * W, W)],
                                    h1a_hbm.at[pl.ds(rb, W)])

                @pl.when(c == 1)
                def _():
                    pltpu.sync_copy(gbuf.at[pl.ds(3 * W, W)],
                                    h1b_hbm.at[pl.ds(rb, W)])

    if last:
        outs = jax.ShapeDtypeStruct((N, D), jnp.float32)
    else:
        outs = (jax.ShapeDtypeStruct((NC * NPAD, HALF), jnp.float32),
                jax.ShapeDtypeStruct((NPAD, HALF), jnp.float32),
                jax.ShapeDtypeStruct((NPAD, HALF), jnp.float32))
    return pl.kernel(
        body,
        out_type=outs,
        mesh=_mesh(),
        compiler_params=_params(),
        scratch_types=[
            pltpu.VMEM((4, 2, EB), jnp.int32),
            pltpu.VMEM((2 * EB, HALF), jnp.float32),
            pltpu.SemaphoreType.DMA,
            pltpu.SemaphoreType.DMA,
            pltpu.SemaphoreType.DMA,
            pltpu.SemaphoreType.DMA,
            pltpu.SemaphoreType.DMA,
            pltpu.SemaphoreType.DMA,
            pltpu.SemaphoreType.DMA,
            pltpu.SemaphoreType.DMA,
            pltpu.SemaphoreType.DMA,
            pltpu.VMEM_SHARED((NACC, HALF), jnp.float32),
        ],
    )


_seg_mid = _make_segsum(last=False)
_seg_last = _make_segsum(last=True)


def _const_body(ones_ref, z16_ref, z32_ref):
    ones_ref[...] = jnp.ones_like(ones_ref)
    z16_ref[...] = jnp.zeros_like(z16_ref)
    z32_ref[...] = jnp.zeros_like(z32_ref)


_const_call = pl.pallas_call(
    _const_body,
    out_shape=(jax.ShapeDtypeStruct((EB, 16), jnp.float32),
               jax.ShapeDtypeStruct((ROWS_PER_TILE, 16), jnp.float32),
               jax.ShapeDtypeStruct((ROWS_PER_TILE, HALF), jnp.float32)),
)


def kernel(emb_table, edge_index, n_id):
    e3 = edge_index.astype(jnp.int32).reshape(2, ECH, EB).transpose(1, 0, 2)
    nidp = jnp.concatenate([n_id.astype(jnp.int32),
                            jnp.zeros((NACC - N,), jnp.int32)])
    ones16, zeros16, zeros32 = _const_call()

    degp = _deg_call(e3, ones16, zeros16)
    x0, h0a, h0b, dis32 = _gs_call(emb_table, nidp, degp)
    x1, h1a, h1b = _seg_mid(h0a, h0b, e3, zeros32, dis32)
    return _seg_last(h1a, h1b, e3, zeros32, dis32, x0, x1)
